# Initial kernel scaffold; baseline (speedup 1.0000x reference)
import functools

import jax
import jax.numpy as jnp
from jax import lax
from jax.experimental import pallas as pl
from jax.experimental.pallas import tpu as pltpu
from jax.experimental.pallas import tpu_sc as plsc

_B, _H, _D = 4096, 200, 128
_NC, _NS = 2, 16
_NW = _NC * _NS
_RPW = _B // _NW
_C1, _C2 = 128, 72
_LANES = 16
_DCH = _D // _LANES


def _pool_body(ids_hbm, w_hbm, table_hbm, out_hbm, idx_v, w_v, rows_v, acc_v, sem):
    wid = lax.axis_index("s") * _NC + lax.axis_index("c")
    base = wid * _RPW

    def row_body(r, carry):
        row = base + r
        pltpu.sync_copy(ids_hbm.at[row], idx_v)
        pltpu.sync_copy(w_hbm.at[row], w_v)
        cp1 = pltpu.async_copy(
            table_hbm.at[idx_v.at[pl.ds(0, _C1)]], rows_v.at[pl.ds(0, _C1)], sem)
        cp2 = pltpu.async_copy(
            table_hbm.at[idx_v.at[pl.ds(_C1, _C2)]], rows_v.at[pl.ds(_C1, _C2)], sem)
        cp1.wait()
        cp2.wait()

        def h_body(h, hc):
            ws, acc = hc
            w = w_v[h]
            acc = tuple(acc[c] + w * rows_v[h, pl.ds(c * _LANES, _LANES)]
                        for c in range(_DCH))
            return ws + w, acc

        zero = jnp.zeros((_LANES,), jnp.float32)
        ws, acc = lax.fori_loop(0, _H, h_body,
                                (jnp.float32(0.0), (zero,) * _DCH))
        inv = 1.0 / (ws + 1e-8)
        for c in range(_DCH):
            acc_v[r, pl.ds(c * _LANES, _LANES)] = acc[c] * inv
        return carry

    lax.fori_loop(0, _RPW, row_body, 0)
    pltpu.sync_copy(acc_v, out_hbm.at[pl.ds(base, _RPW)])


def _sc_pool(history_item_ids, history_item_weights, item_emb):
    mesh = plsc.VectorSubcoreMesh(core_axis_name="c", subcore_axis_name="s")
    kern = pl.kernel(
        _pool_body,
        out_type=jax.ShapeDtypeStruct((_B, _D), jnp.float32),
        mesh=mesh,
        scratch_types=[
            pltpu.VMEM((_H,), jnp.int32),
            pltpu.VMEM((_H,), jnp.float32),
            pltpu.VMEM((_H, _D), jnp.float32),
            pltpu.VMEM((_RPW, _D), jnp.float32),
            pltpu.SemaphoreType.DMA,
        ],
    )
    return kern(history_item_ids, history_item_weights, item_emb)


def _mlp_body(u_ref, p_ref, w1_ref, b1_ref, w2_ref, b2_ref, w3_ref, b3_ref, o_ref):
    x_u = u_ref[...]
    x_p = p_ref[...]
    h = jnp.dot(x_u, w1_ref[0:_D, :], preferred_element_type=jnp.float32)
    h = h + jnp.dot(x_p, w1_ref[_D:2 * _D, :], preferred_element_type=jnp.float32)
    h = jnp.maximum(h + b1_ref[...], 0.0)
    h = jnp.dot(h, w2_ref[...], preferred_element_type=jnp.float32)
    h = jnp.maximum(h + b2_ref[...], 0.0)
    o = jnp.dot(h, w3_ref[...], preferred_element_type=jnp.float32) + b3_ref[...]
    n = jnp.sqrt(jnp.sum(o * o, axis=1, keepdims=True))
    o_ref[...] = o / jnp.maximum(n, 1e-12)


def _tc_mlp(user_id_emb, pooled, W1, b1, W2, b2, W3, b3):
    bt = 512
    grid = (_B // bt,)
    return pl.pallas_call(
        _mlp_body,
        grid=grid,
        in_specs=[
            pl.BlockSpec((bt, _D), lambda i: (i, 0)),
            pl.BlockSpec((bt, _D), lambda i: (i, 0)),
            pl.BlockSpec((2 * _D, 512), lambda i: (0, 0)),
            pl.BlockSpec((1, 512), lambda i: (0, 0)),
            pl.BlockSpec((512, 256), lambda i: (0, 0)),
            pl.BlockSpec((1, 256), lambda i: (0, 0)),
            pl.BlockSpec((256, 64), lambda i: (0, 0)),
            pl.BlockSpec((1, 64), lambda i: (0, 0)),
        ],
        out_specs=pl.BlockSpec((bt, 64), lambda i: (i, 0)),
        out_shape=jax.ShapeDtypeStruct((_B, 64), jnp.float32),
    )(user_id_emb, pooled, W1, b1.reshape(1, -1), W2, b2.reshape(1, -1),
      W3, b3.reshape(1, -1))


@jax.jit
def kernel(user_id_emb, history_item_ids, history_item_weights, item_emb,
           W1, b1, W2, b2, W3, b3):
    pooled = _sc_pool(history_item_ids, history_item_weights, item_emb)
    return _tc_mlp(user_id_emb, pooled, W1, b1, W2, b2, W3, b3)


# speedup vs baseline: 5.3446x; 5.3446x over previous
import functools

import jax
import jax.numpy as jnp
from jax import lax
from jax.experimental import pallas as pl
from jax.experimental.pallas import tpu as pltpu
from jax.experimental.pallas import tpu_sc as plsc

_B, _H, _D = 4096, 200, 128
_NC, _NS = 2, 16
_NW = _NC * _NS
_RPW = _B // _NW
_C1, _C2 = 128, 72
_LANES = 16
_DCH = _D // _LANES


def _pool_body(ids_hbm, w_hbm, table_hbm, out_hbm, idx_v, w_v, rows_v, acc_v, sem):
    wid = lax.axis_index("s") * _NC + lax.axis_index("c")
    base = wid * _RPW

    def row_body(r, carry):
        row = base + r
        pltpu.sync_copy(ids_hbm.at[row], idx_v)
        pltpu.sync_copy(w_hbm.at[row], w_v)
        cp1 = pltpu.async_copy(
            table_hbm.at[idx_v.at[pl.ds(0, _C1)]], rows_v.at[pl.ds(0, _C1)], sem)
        cp2 = pltpu.async_copy(
            table_hbm.at[idx_v.at[pl.ds(_C1, _C2)]], rows_v.at[pl.ds(_C1, _C2)], sem)
        cp1.wait()
        cp2.wait()

        def h_chunk(hc, carry):
            ws, acc = carry
            bh = hc * _LANES
            wchunk = w_v[pl.ds(bh, _LANES)]
            for j in range(_LANES):
                w = wchunk[j]
                ws = ws + w
                acc = tuple(acc[c] + w * rows_v[bh + j, pl.ds(c * _LANES, _LANES)]
                            for c in range(_DCH))
            return ws, acc

        zero = jnp.zeros((_LANES,), jnp.float32)
        nfull = _H // _LANES
        ws, acc = lax.fori_loop(0, nfull, h_chunk,
                                (jnp.float32(0.0), (zero,) * _DCH))
        tb = _H - _LANES
        wtail = w_v[pl.ds(tb, _LANES)]
        for j in range(_LANES - (_H - nfull * _LANES), _LANES):
            w = wtail[j]
            ws = ws + w
            acc = tuple(acc[c] + w * rows_v[tb + j, pl.ds(c * _LANES, _LANES)]
                        for c in range(_DCH))
        denom = jnp.broadcast_to(ws + 1e-8, (_LANES,))
        inv = 1.0 / denom
        for c in range(_DCH):
            acc_v[r, pl.ds(c * _LANES, _LANES)] = acc[c] * inv
        return carry

    lax.fori_loop(0, _RPW, row_body, 0)
    pltpu.sync_copy(acc_v, out_hbm.at[pl.ds(base, _RPW)])


def _sc_pool(history_item_ids, history_item_weights, item_emb):
    mesh = plsc.VectorSubcoreMesh(core_axis_name="c", subcore_axis_name="s")
    kern = pl.kernel(
        _pool_body,
        out_type=jax.ShapeDtypeStruct((_B, _D), jnp.float32),
        mesh=mesh,
        scratch_types=[
            pltpu.VMEM((_H,), jnp.int32),
            pltpu.VMEM((_H,), jnp.float32),
            pltpu.VMEM((_H, _D), jnp.float32),
            pltpu.VMEM((_RPW, _D), jnp.float32),
            pltpu.SemaphoreType.DMA,
        ],
    )
    return kern(history_item_ids, history_item_weights, item_emb)


def _mlp_body(u_ref, p_ref, w1_ref, b1_ref, w2_ref, b2_ref, w3_ref, b3_ref, o_ref):
    x_u = u_ref[...]
    x_p = p_ref[...]
    h = jnp.dot(x_u, w1_ref[0:_D, :], preferred_element_type=jnp.float32)
    h = h + jnp.dot(x_p, w1_ref[_D:2 * _D, :], preferred_element_type=jnp.float32)
    h = jnp.maximum(h + b1_ref[...], 0.0)
    h = jnp.dot(h, w2_ref[...], preferred_element_type=jnp.float32)
    h = jnp.maximum(h + b2_ref[...], 0.0)
    o = jnp.dot(h, w3_ref[...], preferred_element_type=jnp.float32) + b3_ref[...]
    n = jnp.sqrt(jnp.sum(o * o, axis=1, keepdims=True))
    o_ref[...] = o / jnp.maximum(n, 1e-12)


def _tc_mlp(user_id_emb, pooled, W1, b1, W2, b2, W3, b3):
    bt = 512
    grid = (_B // bt,)
    return pl.pallas_call(
        _mlp_body,
        grid=grid,
        in_specs=[
            pl.BlockSpec((bt, _D), lambda i: (i, 0)),
            pl.BlockSpec((bt, _D), lambda i: (i, 0)),
            pl.BlockSpec((2 * _D, 512), lambda i: (0, 0)),
            pl.BlockSpec((1, 512), lambda i: (0, 0)),
            pl.BlockSpec((512, 256), lambda i: (0, 0)),
            pl.BlockSpec((1, 256), lambda i: (0, 0)),
            pl.BlockSpec((256, 64), lambda i: (0, 0)),
            pl.BlockSpec((1, 64), lambda i: (0, 0)),
        ],
        out_specs=pl.BlockSpec((bt, 64), lambda i: (i, 0)),
        out_shape=jax.ShapeDtypeStruct((_B, 64), jnp.float32),
    )(user_id_emb, pooled, W1, b1.reshape(1, -1), W2, b2.reshape(1, -1),
      W3, b3.reshape(1, -1))


@jax.jit
def kernel(user_id_emb, history_item_ids, history_item_weights, item_emb,
           W1, b1, W2, b2, W3, b3):
    pooled = _sc_pool(history_item_ids, history_item_weights, item_emb)
    return _tc_mlp(user_id_emb, pooled, W1, b1, W2, b2, W3, b3)


# speedup vs baseline: 11.7352x; 2.1957x over previous
import functools

import jax
import jax.numpy as jnp
from jax import lax
from jax.experimental import pallas as pl
from jax.experimental.pallas import tpu as pltpu
from jax.experimental.pallas import tpu_sc as plsc

_B, _H, _D = 4096, 200, 128
_NC, _NS = 2, 16
_NW = _NC * _NS
_RPW = _B // _NW
_RPB = _RPW // 2
_C1, _C2 = 128, 72
_LANES = 16
_DCH = _D // _LANES


def _pool_body(ids_hbm, w_hbm, table_hbm, out_hbm, ids_v, w_all, rows_v, acc_v,
               sem0, sem1):
    wid = lax.axis_index("s") * _NC + lax.axis_index("c")
    base = wid * _RPW

    sems = (sem0, sem1)

    def issue(r, b):
        pltpu.async_copy(table_hbm.at[ids_v.at[r, pl.ds(0, _C1)]],
                         rows_v.at[b, pl.ds(0, _C1)], sems[b])
        pltpu.async_copy(table_hbm.at[ids_v.at[r, pl.ds(_C1, _C2)]],
                         rows_v.at[b, pl.ds(_C1, _C2)], sems[b])

    def drain(b):
        pltpu.make_async_copy(table_hbm.at[pl.ds(0, _C1)],
                              rows_v.at[b, pl.ds(0, _C1)], sems[b]).wait()
        pltpu.make_async_copy(table_hbm.at[pl.ds(0, _C2)],
                              rows_v.at[b, pl.ds(_C1, _C2)], sems[b]).wait()

    def compute(r, lr, b):
        rv = rows_v.at[b]

        def h_chunk(hc, carry):
            ws, acc = carry
            bh = hc * _LANES
            wchunk = w_all[lr, pl.ds(bh, _LANES)]
            for j in range(_LANES):
                w = wchunk[j]
                ws = ws + w
                acc = tuple(acc[c] + w * rv[bh + j, pl.ds(c * _LANES, _LANES)]
                            for c in range(_DCH))
            return ws, acc

        zero = jnp.zeros((_LANES,), jnp.float32)
        nfull = _H // _LANES
        ws, acc = lax.fori_loop(0, nfull, h_chunk,
                                (jnp.float32(0.0), (zero,) * _DCH))
        tb = _H - _LANES
        wtail = w_all[lr, pl.ds(tb, _LANES)]
        for j in range(_LANES - (_H - nfull * _LANES), _LANES):
            w = wtail[j]
            ws = ws + w
            acc = tuple(acc[c] + w * rv[tb + j, pl.ds(c * _LANES, _LANES)]
                        for c in range(_DCH))
        denom = jnp.broadcast_to(ws + 1e-8, (_LANES,))
        inv = 1.0 / denom
        for c in range(_DCH):
            acc_v[r, pl.ds(c * _LANES, _LANES)] = acc[c] * inv

    def run_block(rbase):
        pltpu.sync_copy(ids_hbm.at[pl.ds(base + rbase, _RPB)], ids_v)
        pltpu.sync_copy(w_hbm.at[pl.ds(base + rbase, _RPB)], w_all)
        issue(0, 0)

        def outer(i, carry):
            l0 = i * 2
            for b in range(2):
                lr = l0 + b

                @pl.when(lr + 1 < _RPB)
                def _():
                    issue(lr + 1, 1 - b)

                drain(b)
                compute(rbase + lr, lr, b)
            return carry

        lax.fori_loop(0, _RPB // 2, outer, 0)

    for blk in range(_RPW // _RPB):
        run_block(blk * _RPB)
    pltpu.sync_copy(acc_v, out_hbm.at[pl.ds(base, _RPW)])


def _sc_pool(history_item_ids, history_item_weights, item_emb):
    mesh = plsc.VectorSubcoreMesh(core_axis_name="c", subcore_axis_name="s")
    kern = pl.kernel(
        _pool_body,
        out_type=jax.ShapeDtypeStruct((_B, _D), jnp.float32),
        mesh=mesh,
        scratch_types=[
            pltpu.VMEM((_RPB, _H), jnp.int32),
            pltpu.VMEM((_RPB, _H), jnp.float32),
            pltpu.VMEM((2, _H, _D), jnp.float32),
            pltpu.VMEM((_RPW, _D), jnp.float32),
            pltpu.SemaphoreType.DMA,
            pltpu.SemaphoreType.DMA,
        ],
    )
    return kern(history_item_ids, history_item_weights, item_emb)


def _mlp_body(u_ref, p_ref, w1_ref, b1_ref, w2_ref, b2_ref, w3_ref, b3_ref, o_ref):
    x_u = u_ref[...]
    x_p = p_ref[...]
    h = jnp.dot(x_u, w1_ref[0:_D, :], preferred_element_type=jnp.float32)
    h = h + jnp.dot(x_p, w1_ref[_D:2 * _D, :], preferred_element_type=jnp.float32)
    h = jnp.maximum(h + b1_ref[...], 0.0)
    h = jnp.dot(h, w2_ref[...], preferred_element_type=jnp.float32)
    h = jnp.maximum(h + b2_ref[...], 0.0)
    o = jnp.dot(h, w3_ref[...], preferred_element_type=jnp.float32) + b3_ref[...]
    n = jnp.sqrt(jnp.sum(o * o, axis=1, keepdims=True))
    o_ref[...] = o / jnp.maximum(n, 1e-12)


def _tc_mlp(user_id_emb, pooled, W1, b1, W2, b2, W3, b3):
    bt = 512
    grid = (_B // bt,)
    return pl.pallas_call(
        _mlp_body,
        grid=grid,
        in_specs=[
            pl.BlockSpec((bt, _D), lambda i: (i, 0)),
            pl.BlockSpec((bt, _D), lambda i: (i, 0)),
            pl.BlockSpec((2 * _D, 512), lambda i: (0, 0)),
            pl.BlockSpec((1, 512), lambda i: (0, 0)),
            pl.BlockSpec((512, 256), lambda i: (0, 0)),
            pl.BlockSpec((1, 256), lambda i: (0, 0)),
            pl.BlockSpec((256, 64), lambda i: (0, 0)),
            pl.BlockSpec((1, 64), lambda i: (0, 0)),
        ],
        out_specs=pl.BlockSpec((bt, 64), lambda i: (i, 0)),
        out_shape=jax.ShapeDtypeStruct((_B, 64), jnp.float32),
    )(user_id_emb, pooled, W1, b1.reshape(1, -1), W2, b2.reshape(1, -1),
      W3, b3.reshape(1, -1))


@jax.jit
def kernel(user_id_emb, history_item_ids, history_item_weights, item_emb,
           W1, b1, W2, b2, W3, b3):
    pooled = _sc_pool(history_item_ids, history_item_weights, item_emb)
    return _tc_mlp(user_id_emb, pooled, W1, b1, W2, b2, W3, b3)
